# Initial kernel scaffold; baseline (speedup 1.0000x reference)
import jax
import jax.numpy as jnp
from jax.experimental import pallas as pl


def kernel(x, points, count, segment_ids, num_segments):
    raise NotImplementedError("write your pallas kernel here")


# speedup vs baseline: 1.7060x; 1.7060x over previous
import functools

import numpy as np

import jax
import jax.numpy as jnp
from jax import lax
from jax.experimental import pallas as pl
from jax.experimental.pallas import tpu as pltpu
from jax.experimental.pallas import tpu_sc as plsc

N = 320000
D = 128
M = 40000
NW = 32
SEG_PER_W = M // NW
NCH = 5
S = SEG_PER_W // NCH
C = 256
NJ = D // 16

_GDN = lax.GatherDimensionNumbers(
    offset_dims=(), collapsed_slice_dims=(0,), start_index_map=(0,))


def _vgather(v, idx):
    return lax.gather(v, idx[:, None], _GDN, (1,),
                      mode=lax.GatherScatterMode.PROMISE_IN_BOUNDS)


def _make_kernel():
    mesh = plsc.VectorSubcoreMesh(core_axis_name="c", subcore_axis_name="s")

    @functools.partial(
        pl.kernel,
        mesh=mesh,
        out_type=[
            jax.ShapeDtypeStruct((M * D,), jnp.float32),
            jax.ShapeDtypeStruct((M * 16,), jnp.float32),
        ],
        scratch_types=[
            pltpu.VMEM((C * D,), jnp.float32),
            pltpu.VMEM((C * 4,), jnp.float32),
            pltpu.VMEM((C,), jnp.int32),
            pltpu.VMEM((S * D,), jnp.float32),
            pltpu.VMEM((S * 16,), jnp.float32),
            pltpu.VMEM((16,), jnp.int32),
        ],
    )
    def pooled(x_hbm, pc_hbm, ids_hbm, rs_hbm, f_o, o16_o,
               xin, pcin, iin, featb, pbuf, rsv):
        wid = lax.axis_index("s") * 2 + lax.axis_index("c")
        pltpu.sync_copy(rs_hbm.at[wid], rsv)
        rv = rsv[...]

        zero16 = jnp.zeros((16,), jnp.float32)
        neg16 = jnp.full((16,), -jnp.inf, jnp.float32)
        one16 = jnp.ones((16,), jnp.float32)
        idx3 = jnp.full((16,), 3, jnp.int32)

        def _lanes(m):
            return jnp.bitwise_and(lax.iota(jnp.int32, 16) + 4 * m, 15)

        def _m3f():
            f = lax.iota(jnp.int32, 16).astype(jnp.float32)
            return jnp.clip(3.0 - f, 0.0, 1.0)

        for k in range(NCH):
            c0 = wid * SEG_PER_W + k * S
            r_lo = rv[k]
            r_hi = rv[k + 1]

            def zf(z, _):
                featb[pl.ds(z * 16, 16)] = zero16
                return 0
            lax.fori_loop(0, S * D // 16, zf, 0)

            def zp(z, _):
                pbuf[pl.ds(z * 16, 16)] = zero16
                return 0
            lax.fori_loop(0, S, zp, 0)

            b0 = jnp.bitwise_and(r_lo, -16)
            nt = lax.div(r_hi - b0 + (C - 1), C)

            def tbody(t, carry):
                start0 = b0 + t * C
                start = pl.multiple_of(jnp.minimum(start0, N - C), 16)
                pltpu.sync_copy(
                    x_hbm.at[pl.ds(pl.multiple_of(start * D, 2048), C * D)],
                    xin)
                pltpu.sync_copy(
                    pc_hbm.at[pl.ds(pl.multiple_of(start * 4, 64), C * 4)],
                    pcin)
                pltpu.sync_copy(ids_hbm.at[pl.ds(start, C)], iin)
                i_lo = jnp.maximum(r_lo, start0) - start
                i_hi = jnp.minimum(r_hi, start0 + C) - start
                ng = lax.div(i_hi + 15, 16)

                def gbody(g, gc):
                    prev, a, pacc = gc
                    a = list(a)
                    ib = g * 16
                    idv = iin[pl.ds(ib, 16)]
                    pcv = [pcin[pl.ds(ib * 4 + 16 * q, 16)] for q in range(4)]
                    for r in range(16):
                        i = ib + r
                        valid = (i >= i_lo) & (i < i_hi)
                        sid = jnp.where(valid, idv[r], prev)
                        same = sid == prev
                        sl = jnp.maximum(sid - c0, 0)
                        vm = lax.iota(jnp.int32, 16) < valid.astype(
                            jnp.int32) * 16
                        sm = lax.iota(jnp.int32, 16) < same.astype(
                            jnp.int32) * 16
                        for j in range(NJ):
                            rvv = xin[pl.ds(i * D + 16 * j, 16)]
                            rv_eff = jnp.where(vm, rvv, neg16)
                            av = jnp.where(
                                sm, jnp.maximum(a[j], rv_eff), rv_eff)
                            featb[pl.ds(sl * D + 16 * j, 16)] = av
                            a[j] = av
                        dv = _vgather(pcv[r // 4], _lanes(r % 4))
                        csp = _vgather(dv, idx3)
                        cmul = (csp - one16) * _m3f() + one16
                        dvw = dv * cmul
                        dvw = jnp.where(vm, dvw, zero16)
                        pacc = jnp.where(sm, pacc + dvw, dvw)
                        pbuf[pl.ds(sl * 16, 16)] = pacc
                        prev = sid
                    return (prev, tuple(a), pacc)

                return lax.fori_loop(0, ng, gbody, carry)

            init = (jnp.int32(-1), tuple(zero16 for _ in range(NJ)), zero16)
            lax.fori_loop(0, nt, tbody, init)

            def dbody(s, _):
                v = pbuf[pl.ds(s * 16, 16)]
                csp = _vgather(v, idx3)
                csafe = jnp.maximum(csp, one16)
                dvec = (csafe - one16) * _m3f() + one16
                pbuf[pl.ds(s * 16, 16)] = v / dvec
                return 0
            lax.fori_loop(0, S, dbody, 0)

            pltpu.sync_copy(
                featb, f_o.at[pl.ds(pl.multiple_of(c0 * D, 256), S * D)])
            pltpu.sync_copy(
                pbuf, o16_o.at[pl.ds(pl.multiple_of(c0 * 16, 16), S * 16)])

    return pooled


_POOLED = _make_kernel()


def kernel(x, points, count, segment_ids, num_segments):
    del num_segments
    ids32 = segment_ids.astype(jnp.int32)
    pc = jnp.concatenate([points, count], axis=1).reshape(N * 4)
    bounds = jnp.arange(NW * NCH + 1, dtype=jnp.int32) * S
    rs = jnp.searchsorted(ids32, bounds, side="left").astype(jnp.int32)
    idx = jnp.minimum(
        jnp.arange(NW, dtype=jnp.int32)[:, None] * NCH
        + jnp.arange(16, dtype=jnp.int32)[None, :],
        NW * NCH,
    )
    rs2 = rs[idx]
    f_o, o16 = _POOLED(x.reshape(N * D), pc, ids32, rs2)
    down_feats = f_o.reshape(M, D)
    o16 = o16.reshape(M, 16)
    down_points = o16[:, :3]
    down_count = o16[:, 3:4]
    return down_feats, down_points, down_count


# speedup vs baseline: 1.7087x; 1.0016x over previous
import functools

import jax
import jax.numpy as jnp
from jax import lax
from jax.experimental import pallas as pl
from jax.experimental.pallas import tpu as pltpu
from jax.experimental.pallas import tpu_sc as plsc

N = 320000
D = 128
M = 40000
NW = 32
SEG_PER_W = M // NW
NCH = 5
S = SEG_PER_W // NCH
C = 256
NJ = D // 16

_GDN = lax.GatherDimensionNumbers(
    offset_dims=(), collapsed_slice_dims=(0,), start_index_map=(0,))


def _vgather(v, idx):
    return lax.gather(v, idx[:, None], _GDN, (1,),
                      mode=lax.GatherScatterMode.PROMISE_IN_BOUNDS)


def _make_kernel():
    mesh = plsc.VectorSubcoreMesh(core_axis_name="c", subcore_axis_name="s")

    @functools.partial(
        pl.kernel,
        mesh=mesh,
        out_type=[
            jax.ShapeDtypeStruct((M * D,), jnp.float32),
            jax.ShapeDtypeStruct((M * 16,), jnp.float32),
        ],
        scratch_types=[
            pltpu.VMEM((C * D,), jnp.float32),
            pltpu.VMEM((C * 4,), jnp.float32),
            pltpu.VMEM((C,), jnp.int32),
            pltpu.VMEM((S * D,), jnp.float32),
            pltpu.VMEM((S * 16,), jnp.float32),
            pltpu.VMEM((16,), jnp.int32),
        ],
    )
    def pooled(x_hbm, pc_hbm, ids_hbm, rs_hbm, f_o, o16_o,
               xin, pcin, iin, featb, pbuf, rsv):
        wid = lax.axis_index("s") * 2 + lax.axis_index("c")

        zero16 = jnp.zeros((16,), jnp.float32)
        neg16 = jnp.full((16,), -jnp.inf, jnp.float32)
        one16 = jnp.ones((16,), jnp.float32)
        idx3 = jnp.full((16,), 3, jnp.int32)

        def _lanes(m):
            return jnp.bitwise_and(lax.iota(jnp.int32, 16) + 4 * m, 15)

        def _m3f():
            f = lax.iota(jnp.int32, 16).astype(jnp.float32)
            return jnp.clip(3.0 - f, 0.0, 1.0)

        def chunk_body(k, _):
            c0 = wid * SEG_PER_W + k * S
            pltpu.sync_copy(rs_hbm.at[wid, k], rsv)
            rvk = rsv[...]
            r_lo = rvk[0]
            r_hi = rvk[8]

            def zf(z, _):
                featb[pl.ds(z * 16, 16)] = zero16
                return 0
            lax.fori_loop(0, S * D // 16, zf, 0)

            def zp(z, _):
                pbuf[pl.ds(z * 16, 16)] = zero16
                return 0
            lax.fori_loop(0, S, zp, 0)

            b0 = jnp.bitwise_and(r_lo, -16)
            nt = lax.div(r_hi - b0 + (C - 1), C)

            def tbody(t, carry):
                start0 = b0 + t * C
                start = pl.multiple_of(jnp.minimum(start0, N - C), 16)
                pltpu.sync_copy(
                    x_hbm.at[pl.ds(pl.multiple_of(start * D, 2048), C * D)],
                    xin)
                pltpu.sync_copy(
                    pc_hbm.at[pl.ds(pl.multiple_of(start * 4, 64), C * 4)],
                    pcin)
                pltpu.sync_copy(ids_hbm.at[pl.ds(start, C)], iin)
                i_lo = jnp.maximum(r_lo, start0) - start
                i_hi = jnp.minimum(r_hi, start0 + C) - start
                ng = lax.div(i_hi + 15, 16)

                def gbody(g, gc):
                    prev, a, pacc = gc
                    a = list(a)
                    ib = g * 16
                    idv = iin[pl.ds(ib, 16)]
                    pcv = [pcin[pl.ds(ib * 4 + 16 * q, 16)]
                           for q in range(4)]
                    for r in range(16):
                        i = ib + r
                        valid = (i >= i_lo) & (i < i_hi)
                        sid = jnp.where(valid, idv[r], prev)
                        same = sid == prev
                        sl = jnp.maximum(sid - c0, 0)
                        vm = lax.iota(jnp.int32, 16) < valid.astype(
                            jnp.int32) * 16
                        sm = lax.iota(jnp.int32, 16) < same.astype(
                            jnp.int32) * 16
                        sbias = jnp.where(sm, zero16, neg16)
                        vbias = jnp.where(vm, zero16, neg16)
                        for j in range(NJ):
                            rvv = xin[pl.ds(i * D + 16 * j, 16)]
                            a[j] = jnp.maximum(a[j] + sbias, rvv + vbias)
                            featb[pl.ds(sl * D + 16 * j, 16)] = a[j]
                        s01 = jnp.where(sm, one16, zero16)
                        v01 = jnp.where(vm, one16, zero16)
                        dv = _vgather(pcv[r // 4], _lanes(r % 4))
                        csp = _vgather(dv, idx3)
                        cmul = (csp - one16) * _m3f() + one16
                        pacc = pacc * s01 + (dv * cmul) * v01
                        pbuf[pl.ds(sl * 16, 16)] = pacc
                        prev = sid
                    return (prev, tuple(a), pacc)

                return lax.fori_loop(0, ng, gbody, carry)

            init = (jnp.int32(-1), tuple(zero16 for _ in range(NJ)), zero16)
            lax.fori_loop(0, nt, tbody, init)

            def dbody(s, _):
                v = pbuf[pl.ds(s * 16, 16)]
                csp = _vgather(v, idx3)
                csafe = jnp.maximum(csp, one16)
                dvec = (csafe - one16) * _m3f() + one16
                pbuf[pl.ds(s * 16, 16)] = v / dvec
                return 0
            lax.fori_loop(0, S, dbody, 0)

            pltpu.sync_copy(
                featb, f_o.at[pl.ds(pl.multiple_of(c0 * D, 256), S * D)])
            pltpu.sync_copy(
                pbuf, o16_o.at[pl.ds(pl.multiple_of(c0 * 16, 16), S * 16)])
            return 0

        lax.fori_loop(0, NCH, chunk_body, 0)

    return pooled


_POOLED = _make_kernel()


def kernel(x, points, count, segment_ids, num_segments):
    del num_segments
    ids32 = segment_ids.astype(jnp.int32)
    pc = jnp.concatenate([points, count], axis=1).reshape(N * 4)
    bounds = jnp.arange(NW * NCH + 1, dtype=jnp.int32) * S
    rs = jnp.searchsorted(ids32, bounds, side="left").astype(jnp.int32)
    rs_lo = rs[:NW * NCH].reshape(NW, NCH)
    rs_hi = rs[1:NW * NCH + 1].reshape(NW, NCH)
    lane = jnp.arange(16, dtype=jnp.int32)
    rs2 = jnp.where(lane[None, None, :] < 8,
                    rs_lo[:, :, None], rs_hi[:, :, None])
    f_o, o16 = _POOLED(x.reshape(N * D), pc, ids32, rs2)
    down_feats = f_o.reshape(M, D)
    o16 = o16.reshape(M, 16)
    down_points = o16[:, :3]
    down_count = o16[:, 3:4]
    return down_feats, down_points, down_count


# speedup vs baseline: 1.9123x; 1.1191x over previous
import functools

import jax
import jax.numpy as jnp
from jax import lax
from jax.experimental import pallas as pl
from jax.experimental.pallas import tpu as pltpu
from jax.experimental.pallas import tpu_sc as plsc

N = 320000
D = 128
M = 40000
NW = 32
SEG_PER_W = M // NW
NCH = 5
S = SEG_PER_W // NCH
C = 256
NJ = D // 16

_GDN = lax.GatherDimensionNumbers(
    offset_dims=(), collapsed_slice_dims=(0,), start_index_map=(0,))


def _vgather(v, idx):
    return lax.gather(v, idx[:, None], _GDN, (1,),
                      mode=lax.GatherScatterMode.PROMISE_IN_BOUNDS)


def _make_kernel():
    mesh = plsc.VectorSubcoreMesh(core_axis_name="c", subcore_axis_name="s")

    @functools.partial(
        pl.kernel,
        mesh=mesh,
        out_type=[
            jax.ShapeDtypeStruct((M * D,), jnp.float32),
            jax.ShapeDtypeStruct((M * 16,), jnp.float32),
        ],
        scratch_types=[
            pltpu.VMEM((C * D,), jnp.float32),
            pltpu.VMEM((C * 4,), jnp.float32),
            pltpu.VMEM((C,), jnp.int32),
            pltpu.VMEM((C * D,), jnp.float32),
            pltpu.VMEM((C * 4,), jnp.float32),
            pltpu.VMEM((C,), jnp.int32),
            pltpu.VMEM((S * D,), jnp.float32),
            pltpu.VMEM((S * 16,), jnp.float32),
            pltpu.VMEM((16,), jnp.int32),
            pltpu.SemaphoreType.DMA,
            pltpu.SemaphoreType.DMA,
        ],
    )
    def pooled(x_hbm, pc_hbm, ids_hbm, rs_hbm, f_o, o16_o,
               xinA, pcinA, iinA, xinB, pcinB, iinB,
               featb, pbuf, rsv, semA, semB):
        wid = lax.axis_index("s") * 2 + lax.axis_index("c")

        zero16 = jnp.zeros((16,), jnp.float32)
        neg16 = jnp.full((16,), -jnp.inf, jnp.float32)
        one16 = jnp.ones((16,), jnp.float32)
        idx3 = jnp.full((16,), 3, jnp.int32)

        def _lanes(m):
            return jnp.bitwise_and(lax.iota(jnp.int32, 16) + 4 * m, 15)

        def _m3f():
            f = lax.iota(jnp.int32, 16).astype(jnp.float32)
            return jnp.clip(3.0 - f, 0.0, 1.0)

        def chunk_body(k, _):
            c0 = wid * SEG_PER_W + k * S
            pltpu.sync_copy(rs_hbm.at[wid, k], rsv)
            rvk = rsv[...]
            r_lo = rvk[0]
            r_hi = rvk[8]

            def zf(z, _):
                featb[pl.ds(z * 16, 16)] = zero16
                return 0
            lax.fori_loop(0, S * D // 16, zf, 0)

            def zp(z, _):
                pbuf[pl.ds(z * 16, 16)] = zero16
                return 0
            lax.fori_loop(0, S, zp, 0)

            b0 = jnp.bitwise_and(r_lo, -16)
            nt = lax.div(r_hi - b0 + (C - 1), C)

            def _start(t):
                return pl.multiple_of(
                    jnp.minimum(b0 + t * C, N - C), 16)

            def issue3(bufs, sem, t):
                xin, pcin, iin = bufs
                start = _start(t)
                pltpu.async_copy(
                    x_hbm.at[pl.ds(pl.multiple_of(start * D, 2048), C * D)],
                    xin, sem)
                pltpu.async_copy(
                    pc_hbm.at[pl.ds(pl.multiple_of(start * 4, 64), C * 4)],
                    pcin, sem)
                pltpu.async_copy(ids_hbm.at[pl.ds(start, C)], iin, sem)

            def drain3(bufs, sem):
                xin, pcin, iin = bufs
                pltpu.make_async_copy(
                    x_hbm.at[pl.ds(0, C * D)], xin, sem).wait()
                pltpu.make_async_copy(
                    pc_hbm.at[pl.ds(0, C * 4)], pcin, sem).wait()
                pltpu.make_async_copy(
                    ids_hbm.at[pl.ds(0, C)], iin, sem).wait()

            def win(bufs, t, carry):
                xin, pcin, iin = bufs
                start0 = b0 + t * C
                start = _start(t)
                i_lo = jnp.maximum(r_lo, start0) - start
                i_hi = jnp.minimum(r_hi, start0 + C) - start
                ng = lax.div(i_hi + 15, 16)

                def gbody(g, gc):
                    prev, a, pacc = gc
                    a = list(a)
                    ib = g * 16
                    idv = iin[pl.ds(ib, 16)]
                    pcv = [pcin[pl.ds(ib * 4 + 16 * q, 16)]
                           for q in range(4)]
                    for r in range(16):
                        i = ib + r
                        valid = (i >= i_lo) & (i < i_hi)
                        sid = jnp.where(valid, idv[r], prev)
                        same = sid == prev
                        sl = jnp.maximum(sid - c0, 0)
                        vm = lax.iota(jnp.int32, 16) < valid.astype(
                            jnp.int32) * 16
                        sm = lax.iota(jnp.int32, 16) < same.astype(
                            jnp.int32) * 16
                        sbias = jnp.where(sm, zero16, neg16)
                        vbias = jnp.where(vm, zero16, neg16)
                        for j in range(NJ):
                            rvv = xin[pl.ds(i * D + 16 * j, 16)]
                            a[j] = jnp.maximum(a[j] + sbias, rvv + vbias)
                            featb[pl.ds(sl * D + 16 * j, 16)] = a[j]
                        s01 = jnp.where(sm, one16, zero16)
                        v01 = jnp.where(vm, one16, zero16)
                        dv = _vgather(pcv[r // 4], _lanes(r % 4))
                        csp = _vgather(dv, idx3)
                        cmul = (csp - one16) * _m3f() + one16
                        pacc = pacc * s01 + (dv * cmul) * v01
                        pbuf[pl.ds(sl * 16, 16)] = pacc
                        prev = sid
                    return (prev, tuple(a), pacc)

                return lax.fori_loop(0, ng, gbody, carry)

            bufsA = (xinA, pcinA, iinA)
            bufsB = (xinB, pcinB, iinB)
            issue3(bufsA, semA, 0)
            ntp = lax.div(jnp.maximum(nt, 1) + 1, 2)

            def pair(tp, carry):
                drain3(bufsA, semA)
                issue3(bufsB, semB, 2 * tp + 1)
                carry = win(bufsA, 2 * tp, carry)
                drain3(bufsB, semB)
                issue3(bufsA, semA, 2 * tp + 2)
                return win(bufsB, 2 * tp + 1, carry)

            init = (jnp.int32(-1), tuple(zero16 for _ in range(NJ)), zero16)
            lax.fori_loop(0, ntp, pair, init)
            drain3(bufsA, semA)

            def dbody(s, _):
                v = pbuf[pl.ds(s * 16, 16)]
                csp = _vgather(v, idx3)
                csafe = jnp.maximum(csp, one16)
                dvec = (csafe - one16) * _m3f() + one16
                pbuf[pl.ds(s * 16, 16)] = v / dvec
                return 0
            lax.fori_loop(0, S, dbody, 0)

            pltpu.sync_copy(
                featb, f_o.at[pl.ds(pl.multiple_of(c0 * D, 256), S * D)])
            pltpu.sync_copy(
                pbuf, o16_o.at[pl.ds(pl.multiple_of(c0 * 16, 16), S * 16)])
            return 0

        lax.fori_loop(0, NCH, chunk_body, 0)

    return pooled


_POOLED = _make_kernel()


def kernel(x, points, count, segment_ids, num_segments):
    del num_segments
    ids32 = segment_ids.astype(jnp.int32)
    pc = jnp.concatenate([points, count], axis=1).reshape(N * 4)
    bounds = jnp.arange(NW * NCH + 1, dtype=jnp.int32) * S
    rs = jnp.searchsorted(ids32, bounds, side="left").astype(jnp.int32)
    rs_lo = rs[:NW * NCH].reshape(NW, NCH)
    rs_hi = rs[1:NW * NCH + 1].reshape(NW, NCH)
    lane = jnp.arange(16, dtype=jnp.int32)
    rs2 = jnp.where(lane[None, None, :] < 8,
                    rs_lo[:, :, None], rs_hi[:, :, None])
    f_o, o16 = _POOLED(x.reshape(N * D), pc, ids32, rs2)
    down_feats = f_o.reshape(M, D)
    o16 = o16.reshape(M, 16)
    down_points = o16[:, :3]
    down_count = o16[:, 3:4]
    return down_feats, down_points, down_count


# speedup vs baseline: 3.0287x; 1.5838x over previous
import functools

import jax
import jax.numpy as jnp
from jax import lax
from jax.experimental import pallas as pl
from jax.experimental.pallas import tpu as pltpu
from jax.experimental.pallas import tpu_sc as plsc

N = 320000
D = 128
M = 40000
NW = 32
SEG_PER_W = M // NW
NCH = 5
S = SEG_PER_W // NCH
C = 256
NJ = D // 16

_GDN = lax.GatherDimensionNumbers(
    offset_dims=(), collapsed_slice_dims=(0,), start_index_map=(0,))


def _vgather(v, idx):
    return lax.gather(v, idx[:, None], _GDN, (1,),
                      mode=lax.GatherScatterMode.PROMISE_IN_BOUNDS)


def _make_kernel():
    mesh = plsc.VectorSubcoreMesh(core_axis_name="c", subcore_axis_name="s")

    @functools.partial(
        pl.kernel,
        mesh=mesh,
        out_type=[
            jax.ShapeDtypeStruct((M * D,), jnp.float32),
            jax.ShapeDtypeStruct((M * 16,), jnp.float32),
        ],
        scratch_types=[
            pltpu.VMEM((C * D,), jnp.float32),
            pltpu.VMEM((C * 4,), jnp.float32),
            pltpu.VMEM((C,), jnp.int32),
            pltpu.VMEM((C * D,), jnp.float32),
            pltpu.VMEM((C * 4,), jnp.float32),
            pltpu.VMEM((C,), jnp.int32),
            pltpu.VMEM((S * D,), jnp.float32),
            pltpu.VMEM((S * 16,), jnp.float32),
            pltpu.VMEM((16,), jnp.int32),
            pltpu.SemaphoreType.DMA,
            pltpu.SemaphoreType.DMA,
        ],
    )
    def pooled(x_hbm, pc_hbm, ids_hbm, rs_hbm, f_o, o16_o,
               xinA, pcinA, iinA, xinB, pcinB, iinB,
               featb, pbuf, rsv, semA, semB):
        wid = lax.axis_index("s") * 2 + lax.axis_index("c")

        zero16 = jnp.zeros((16,), jnp.float32)
        neg16 = jnp.full((16,), -jnp.inf, jnp.float32)
        one16 = jnp.ones((16,), jnp.float32)
        idx3 = jnp.full((16,), 3, jnp.int32)

        def _lanes(m):
            return jnp.bitwise_and(lax.iota(jnp.int32, 16) + 4 * m, 15)

        def _m3f():
            f = lax.iota(jnp.int32, 16).astype(jnp.float32)
            return jnp.clip(3.0 - f, 0.0, 1.0)

        def chunk_body(k, _):
            c0 = wid * SEG_PER_W + k * S
            pltpu.sync_copy(rs_hbm.at[wid, k], rsv)
            rvk = rsv[...]
            r_lo = rvk[0]
            r_hi = rvk[8]

            def zf(z, _):
                featb[pl.ds(z * 16, 16)] = zero16
                return 0
            lax.fori_loop(0, S * D // 16, zf, 0)

            def zp(z, _):
                pbuf[pl.ds(z * 16, 16)] = zero16
                return 0
            lax.fori_loop(0, S, zp, 0)

            b0 = jnp.bitwise_and(r_lo, -16)
            nt = lax.div(r_hi - b0 + (C - 1), C)

            def _start(t):
                return pl.multiple_of(
                    jnp.minimum(b0 + t * C, N - C), 16)

            def issue3(bufs, sem, t):
                xin, pcin, iin = bufs
                start = _start(t)
                pltpu.async_copy(
                    x_hbm.at[pl.ds(pl.multiple_of(start * D, 2048), C * D)],
                    xin, sem)
                pltpu.async_copy(
                    pc_hbm.at[pl.ds(pl.multiple_of(start * 4, 64), C * 4)],
                    pcin, sem)
                pltpu.async_copy(ids_hbm.at[pl.ds(start, C)], iin, sem)

            def drain3(bufs, sem):
                xin, pcin, iin = bufs
                pltpu.make_async_copy(
                    x_hbm.at[pl.ds(0, C * D)], xin, sem).wait()
                pltpu.make_async_copy(
                    pc_hbm.at[pl.ds(0, C * 4)], pcin, sem).wait()
                pltpu.make_async_copy(
                    ids_hbm.at[pl.ds(0, C)], iin, sem).wait()

            def win(bufs, t, carry):
                xin, pcin, iin = bufs
                start0 = b0 + t * C
                start = _start(t)
                i_lo = jnp.maximum(r_lo, start0) - start
                i_hi = jnp.minimum(r_hi, start0 + C) - start
                ng = lax.div(i_hi + 15, 16)

                def gbody(g, gc):
                    prev, a, pacc = gc
                    a = list(a)
                    ib = g * 16
                    idv = iin[pl.ds(ib, 16)]
                    pcv = [pcin[pl.ds(ib * 4 + 16 * q, 16)]
                           for q in range(4)]
                    for r in range(16):
                        i = ib + r
                        valid = (i >= i_lo) & (i < i_hi)
                        sid = jnp.where(valid, idv[r], prev)
                        same = sid == prev
                        vm = lax.iota(jnp.int32, 16) < valid.astype(
                            jnp.int32) * 16
                        sm = lax.iota(jnp.int32, 16) < same.astype(
                            jnp.int32) * 16
                        aa, pp = tuple(a), pacc

                        @pl.when(jnp.logical_not(same) & (prev >= c0))
                        def _():
                            spl = jnp.maximum(prev - c0, 0)
                            for j in range(NJ):
                                featb[pl.ds(spl * D + 16 * j, 16)] = aa[j]
                            pbuf[pl.ds(spl * 16, 16)] = pp

                        sbias = jnp.where(sm, zero16, neg16)
                        vbias = jnp.where(vm, zero16, neg16)
                        for j in range(NJ):
                            rvv = xin[pl.ds(i * D + 16 * j, 16)]
                            a[j] = jnp.maximum(a[j] + sbias, rvv + vbias)
                        s01 = jnp.where(sm, one16, zero16)
                        v01 = jnp.where(vm, one16, zero16)
                        dv = _vgather(pcv[r // 4], _lanes(r % 4))
                        csp = _vgather(dv, idx3)
                        cmul = (csp - one16) * _m3f() + one16
                        pacc = pacc * s01 + (dv * cmul) * v01
                        prev = sid
                    return (prev, tuple(a), pacc)

                return lax.fori_loop(0, ng, gbody, carry)

            bufsA = (xinA, pcinA, iinA)
            bufsB = (xinB, pcinB, iinB)
            issue3(bufsA, semA, 0)
            ntp = lax.div(jnp.maximum(nt, 1) + 1, 2)

            def pair(tp, carry):
                drain3(bufsA, semA)
                issue3(bufsB, semB, 2 * tp + 1)
                carry = win(bufsA, 2 * tp, carry)
                drain3(bufsB, semB)
                issue3(bufsA, semA, 2 * tp + 2)
                return win(bufsB, 2 * tp + 1, carry)

            init = (jnp.int32(-1), tuple(zero16 for _ in range(NJ)), zero16)
            fprev, fa, fpacc = lax.fori_loop(0, ntp, pair, init)
            drain3(bufsA, semA)

            @pl.when(fprev >= c0)
            def _():
                spl = jnp.maximum(fprev - c0, 0)
                for j in range(NJ):
                    featb[pl.ds(spl * D + 16 * j, 16)] = fa[j]
                pbuf[pl.ds(spl * 16, 16)] = fpacc

            def dbody(s, _):
                v = pbuf[pl.ds(s * 16, 16)]
                csp = _vgather(v, idx3)
                csafe = jnp.maximum(csp, one16)
                dvec = (csafe - one16) * _m3f() + one16
                pbuf[pl.ds(s * 16, 16)] = v / dvec
                return 0
            lax.fori_loop(0, S, dbody, 0)

            pltpu.sync_copy(
                featb, f_o.at[pl.ds(pl.multiple_of(c0 * D, 256), S * D)])
            pltpu.sync_copy(
                pbuf, o16_o.at[pl.ds(pl.multiple_of(c0 * 16, 16), S * 16)])
            return 0

        lax.fori_loop(0, NCH, chunk_body, 0)

    return pooled


_POOLED = _make_kernel()


def kernel(x, points, count, segment_ids, num_segments):
    del num_segments
    ids32 = segment_ids.astype(jnp.int32)
    pc = jnp.concatenate([points, count], axis=1).reshape(N * 4)
    bounds = jnp.arange(NW * NCH + 1, dtype=jnp.int32) * S
    rs = jnp.searchsorted(ids32, bounds, side="left").astype(jnp.int32)
    rs_lo = rs[:NW * NCH].reshape(NW, NCH)
    rs_hi = rs[1:NW * NCH + 1].reshape(NW, NCH)
    lane = jnp.arange(16, dtype=jnp.int32)
    rs2 = jnp.where(lane[None, None, :] < 8,
                    rs_lo[:, :, None], rs_hi[:, :, None])
    f_o, o16 = _POOLED(x.reshape(N * D), pc, ids32, rs2)
    down_feats = f_o.reshape(M, D)
    o16 = o16.reshape(M, 16)
    down_points = o16[:, :3]
    down_count = o16[:, 3:4]
    return down_feats, down_points, down_count
